# baseline (speedup 1.0000x reference)
import functools
import math

import jax
import jax.numpy as jnp
from jax import lax
from jax.experimental import pallas as pl
from jax.experimental.pallas import tpu as pltpu

_G = 2.0
_NCLS = 80
_B, _A, _H, _W = 16, 3, 80, 80
_C = 5 + _NCLS
_HW = _H * _W
_N = _B * _A * _HW
_CHUNK = 4096
_NSTEP = _N // _CHUNK
_EPS = 1e-7


def _bce(p, t):
    return jnp.maximum(p, 0.0) - p * t + jnp.log(1.0 + jnp.exp(-jnp.abs(p)))


def _atan_pos(x):
    big = x > 1.0
    r = jnp.where(big, 1.0 / jnp.maximum(x, 1e-30), x)
    r2 = r * r
    y = r * (0.9998660 + r2 * (-0.3302995 + r2 * (0.1801410 + r2 * (-0.0851330 + r2 * 0.0208351))))
    return jnp.where(big, (math.pi / 2.0) - y, y)


def _sigmoid(x):
    return 1.0 / (1.0 + jnp.exp(-x))


def _body(p_ref, t_ref, m_ref, awh_ref, out_ref, acc_ref):
    i = pl.program_id(0)

    @pl.when(i == 0)
    def _init():
        for j in range(10):
            acc_ref[j] = 0.0

    g = p_ref[...]
    t = t_ref[...]
    m = m_ref[...]

    acc_ref[9] += jnp.sum(_bce(g[:, 4:5], t[:, 4:5]))

    rows = i * _CHUNK + lax.broadcasted_iota(jnp.int32, (_CHUNK, 1), 0)
    aid = (rows // _HW) % _A

    aw = jnp.where(aid == 0, awh_ref[0, 0, 0],
                   jnp.where(aid == 1, awh_ref[1, 0, 0], awh_ref[2, 0, 0]))
    ah = jnp.where(aid == 0, awh_ref[0, 0, 1],
                   jnp.where(aid == 1, awh_ref[1, 0, 1], awh_ref[2, 0, 1]))

    sb = _sigmoid(g[:, 0:4])
    px = sb[:, 0:1] * _G - (_G - 1.0) / 2.0
    py = sb[:, 1:2] * _G - (_G - 1.0) / 2.0
    pw = (sb[:, 2:3] * _G) ** 2 * aw
    ph = (sb[:, 3:4] * _G) ** 2 * ah
    tx, ty = t[:, 0:1], t[:, 1:2]
    tw, th = t[:, 2:3], t[:, 3:4]

    px1, px2 = px - pw * 0.5, px + pw * 0.5
    py1, py2 = py - ph * 0.5, py + ph * 0.5
    tx1, tx2 = tx - tw * 0.5, tx + tw * 0.5
    ty1, ty2 = ty - th * 0.5, ty + th * 0.5
    iw = jnp.maximum(jnp.minimum(px2, tx2) - jnp.maximum(px1, tx1), 0.0)
    ih = jnp.maximum(jnp.minimum(py2, ty2) - jnp.maximum(py1, ty1), 0.0)
    inter = iw * ih
    union = pw * ph + tw * th - inter + _EPS
    iou = inter / union
    cw = jnp.maximum(px2, tx2) - jnp.minimum(px1, tx1)
    ch = jnp.maximum(py2, ty2) - jnp.minimum(py1, ty1)
    c2 = cw * cw + ch * ch + _EPS
    rho2 = (px - tx) ** 2 + (py - ty) ** 2
    dv = _atan_pos(tw / (th + _EPS)) - _atan_pos(pw / (ph + _EPS))
    v = (4.0 / (math.pi ** 2)) * dv * dv
    alpha = v / (1.0 - iou + v + _EPS)
    ciou_loss = (1.0 - (iou - rho2 / c2 - alpha * v)) * m

    clsbce = _bce(g[:, 5:], t[:, 5:]) * m

    for k in range(_A):
        sel = aid == k
        acc_ref[k] += jnp.sum(jnp.where(sel, ciou_loss, 0.0))
        acc_ref[3 + k] += jnp.sum(jnp.where(sel, clsbce, 0.0))
        acc_ref[6 + k] += jnp.sum(jnp.where(sel, m, 0.0))

    @pl.when(i == _NSTEP - 1)
    def _final():
        tot = acc_ref[9] / _N
        for k in range(_A):
            cntk = acc_ref[6 + k]
            safe = jnp.maximum(cntk, 1.0)
            contrib = acc_ref[k] / safe + acc_ref[3 + k] / (safe * _NCLS)
            tot += jnp.where(cntk > 0.0, contrib, 0.0)
        out_ref[0, 0] = tot


@jax.jit
def _yolo_loss(pred2, tgt2, mask2, awh):
    out = pl.pallas_call(
        _body,
        grid=(_NSTEP,),
        in_specs=[
            pl.BlockSpec((_CHUNK, _C), lambda i: (i, 0)),
            pl.BlockSpec((_CHUNK, _C), lambda i: (i, 0)),
            pl.BlockSpec((_CHUNK, 1), lambda i: (i, 0)),
            pl.BlockSpec((_A, 1, 2), lambda i: (0, 0, 0), memory_space=pltpu.SMEM),
        ],
        out_specs=pl.BlockSpec((1, 1), lambda i: (0, 0), memory_space=pltpu.SMEM),
        out_shape=jax.ShapeDtypeStruct((1, 1), jnp.float32),
        scratch_shapes=[pltpu.SMEM((16,), jnp.float32)],
    )(pred2, tgt2, mask2, awh)
    return out[0, 0]


def kernel(predictions, targets_in_grid, targets_masks, anchors):
    pred2 = predictions.transpose(0, 1, 3, 4, 2).reshape(_N, _C)
    tgt2 = targets_in_grid.transpose(0, 1, 3, 4, 2).reshape(_N, _C)
    mask2 = targets_masks.reshape(_N, 1).astype(jnp.float32)
    awh = anchors[:, 2:4].reshape(_A, 1, 2)
    return _yolo_loss(pred2, tgt2, mask2, awh)


# speedup vs baseline: 3.4930x; 3.4930x over previous
import functools
import math

import jax
import jax.numpy as jnp
from jax.experimental import pallas as pl
from jax.experimental.pallas import tpu as pltpu

_G = 2.0
_NCLS = 80
_B, _A, _H, _W = 16, 3, 80, 80
_C = 5 + _NCLS
_HW = _H * _W
_EPS = 1e-7


def _bce(p, t):
    return jnp.maximum(p, 0.0) - p * t + jnp.log(1.0 + jnp.exp(-jnp.abs(p)))


def _atan_pos(x):
    big = x > 1.0
    r = jnp.where(big, 1.0 / jnp.maximum(x, 1e-30), x)
    r2 = r * r
    y = r * (0.9998660 + r2 * (-0.3302995 + r2 * (0.1801410 + r2 * (-0.0851330 + r2 * 0.0208351))))
    return jnp.where(big, (math.pi / 2.0) - y, y)


def _sigmoid(x):
    return 1.0 / (1.0 + jnp.exp(-x))


def _dense_body(pred_ref, tgt_ref, mask_ref, awh_ref, out_ref, acc_ref):
    a = pl.program_id(0)
    b = pl.program_id(1)

    p = pred_ref[0]
    t = tgt_ref[0]
    m = mask_ref[0]

    obj_part = jnp.sum(_bce(p[4:5, :], t[4:5, :]))

    aw = awh_ref[0, 0, 0]
    ah = awh_ref[0, 0, 1]
    sb = _sigmoid(p[0:4, :])
    px = sb[0:1, :] * _G - (_G - 1.0) / 2.0
    py = sb[1:2, :] * _G - (_G - 1.0) / 2.0
    pw = (sb[2:3, :] * _G) ** 2 * aw
    ph = (sb[3:4, :] * _G) ** 2 * ah
    tx, ty, tw, th = t[0:1, :], t[1:2, :], t[2:3, :], t[3:4, :]

    px1, px2 = px - pw * 0.5, px + pw * 0.5
    py1, py2 = py - ph * 0.5, py + ph * 0.5
    tx1, tx2 = tx - tw * 0.5, tx + tw * 0.5
    ty1, ty2 = ty - th * 0.5, ty + th * 0.5
    iw = jnp.maximum(jnp.minimum(px2, tx2) - jnp.maximum(px1, tx1), 0.0)
    ih = jnp.maximum(jnp.minimum(py2, ty2) - jnp.maximum(py1, ty1), 0.0)
    inter = iw * ih
    union = pw * ph + tw * th - inter + _EPS
    iou = inter / union
    cw = jnp.maximum(px2, tx2) - jnp.minimum(px1, tx1)
    ch = jnp.maximum(py2, ty2) - jnp.minimum(py1, ty1)
    c2 = cw * cw + ch * ch + _EPS
    rho2 = (px - tx) ** 2 + (py - ty) ** 2
    dv = _atan_pos(tw / (th + _EPS)) - _atan_pos(pw / (ph + _EPS))
    v = (4.0 / (math.pi ** 2)) * dv * dv
    alpha = v / (1.0 - iou + v + _EPS)
    ciou_loss = 1.0 - (iou - rho2 / c2 - alpha * v)
    box_part = jnp.sum(ciou_loss * m)

    cls_part = jnp.sum(_bce(p[5:, :], t[5:, :]) * m)
    cnt_part = jnp.sum(m)

    first = jnp.logical_and(a == 0, b == 0)

    @pl.when(first)
    def _init():
        acc_ref[0] = 0.0
        acc_ref[1] = 0.0

    @pl.when(b == 0)
    def _reset():
        acc_ref[2] = 0.0
        acc_ref[3] = 0.0
        acc_ref[4] = 0.0

    acc_ref[0] += obj_part
    acc_ref[2] += box_part
    acc_ref[3] += cls_part
    acc_ref[4] += cnt_part

    @pl.when(b == _B - 1)
    def _fold_anchor():
        cnt = acc_ref[4]
        safe = jnp.maximum(cnt, 1.0)
        contrib = acc_ref[2] / safe + acc_ref[3] / (safe * _NCLS)
        acc_ref[1] += jnp.where(cnt > 0.0, contrib, 0.0)

    @pl.when(jnp.logical_and(a == _A - 1, b == _B - 1))
    def _final():
        out_ref[0, 0] = acc_ref[1] + acc_ref[0] / (_B * _A * _H * _W)


@functools.partial(jax.jit, static_argnames=())
def _yolo_loss_dense(pred3, tgt3, mask3, awh):
    grid = (_A, _B)
    out = pl.pallas_call(
        _dense_body,
        grid=grid,
        in_specs=[
            pl.BlockSpec((1, _C, _HW), lambda a, b: (b * _A + a, 0, 0)),
            pl.BlockSpec((1, _C, _HW), lambda a, b: (b * _A + a, 0, 0)),
            pl.BlockSpec((1, 1, _HW), lambda a, b: (b * _A + a, 0, 0)),
            pl.BlockSpec((1, 1, 2), lambda a, b: (a, 0, 0), memory_space=pltpu.SMEM),
        ],
        out_specs=pl.BlockSpec((1, 1), lambda a, b: (0, 0), memory_space=pltpu.SMEM),
        out_shape=jax.ShapeDtypeStruct((1, 1), jnp.float32),
        scratch_shapes=[pltpu.SMEM((8,), jnp.float32)],
    )(pred3, tgt3, mask3, awh)
    return out[0, 0]


def kernel(predictions, targets_in_grid, targets_masks, anchors):
    pred3 = predictions.reshape(_B * _A, _C, _HW)
    tgt3 = targets_in_grid.reshape(_B * _A, _C, _HW)
    mask3 = targets_masks.astype(jnp.float32).reshape(_B * _A, 1, _HW)
    awh = anchors[:, 2:4].reshape(_A, 1, 2)
    return _yolo_loss_dense(pred3, tgt3, mask3, awh)
